# baseline (speedup 1.0000x reference)
import jax
import jax.numpy as jnp
from jax import lax
from jax.experimental import pallas as pl
from jax.experimental.pallas import tpu as pltpu
from jax.experimental.pallas import tpu_sc as plsc

N_PTS = 50000
N_CL = 5000
E = 600000
D = 128
NG = 16
NC_OUT = 10

SC_CORES = 2
SC_SUBCORES = 16
NW = SC_CORES * SC_SUBCORES

RANGE = 157
N_ROWS = NW * RANGE
ZROW = N_PTS
CW = 8

CHUNK = 128
NCH = 160
CAP = NCH * CHUNK

CNT_ROWS = 160
SUB_ROWS = 320
RB = 1256
NBLK = N_ROWS // RB


def _sc_edge_aggregate_body(x_hbm, es_hbm, ed_hbm, zsum_hbm, out_sum, out_cnt,
                            src_idx_v, dst_idx_v, rows_v, idx_buf, ones_v,
                            acc_sum, sem0):
    cid = lax.axis_index("c")
    sid = lax.axis_index("s")
    wid = cid * SC_SUBCORES + sid

    pltpu.sync_copy(es_hbm.at[wid], src_idx_v)
    pltpu.sync_copy(ed_hbm.at[wid], dst_idx_v)

    r0 = pl.multiple_of(jnp.minimum(sid * SUB_ROWS, N_ROWS - SUB_ROWS), 8)
    pltpu.sync_copy(zsum_hbm.at[pl.ds(r0, SUB_ROWS)],
                    acc_sum.at[pl.ds(r0, SUB_ROWS)])
    plsc.subcore_barrier()

    def loop_body(jj, carry):
        pltpu.async_copy(x_hbm.at[src_idx_v.at[jj]], rows_v, sem0).wait()
        for k in range(CHUNK // 16):
            idx_buf[pl.ds(k * 16, 16)] = dst_idx_v[jj, pl.ds(k * 16, 16)]
        pltpu.sync_copy(rows_v, acc_sum.at[idx_buf], add=True)
        return carry

    lax.fori_loop(0, NCH, loop_body, 0)

    plsc.subcore_barrier()
    pltpu.sync_copy(acc_sum.at[pl.ds(r0, SUB_ROWS)],
                    out_sum.at[cid, pl.ds(r0, SUB_ROWS)])
    plsc.subcore_barrier()

    pltpu.sync_copy(zsum_hbm.at[pl.ds(r0, SUB_ROWS)],
                    acc_sum.at[pl.ds(r0, SUB_ROWS)])
    for r in range(CHUNK):
        for k in range(D // 16):
            ones_v[r, pl.ds(k * 16, 16)] = jnp.ones((16,), jnp.float32)
    plsc.subcore_barrier()

    def cnt_body(jj, carry):
        for k in range(CHUNK // 16):
            idx_buf[pl.ds(k * 16, 16)] = dst_idx_v[jj, pl.ds(k * 16, 16)]
        pltpu.sync_copy(ones_v, acc_sum.at[idx_buf], add=True)
        return carry

    lax.fori_loop(0, NCH, cnt_body, 0)

    plsc.subcore_barrier()
    pltpu.sync_copy(acc_sum.at[pl.ds(r0, SUB_ROWS)],
                    out_cnt.at[cid, pl.ds(r0, SUB_ROWS)])


def _tc_body(sum_ref, cnt3_ref, padc_ref, xc_ref, b3_ref,
             w0, ws0, bb0, w1, ws1, bb1, w2, ws2, bb2, wl, bl,
             out_ref, pooled):
    i = pl.program_id(0)
    s = sum_ref[0] + sum_ref[1]
    c = (cnt3_ref[0, :, 0:1] + cnt3_ref[1, :, 0:1]) - padc_ref[0]
    agg = s / jnp.maximum(c, 1.0)
    x = xc_ref[...]

    def layer(a, h, w, ws, bb):
        y = (jnp.dot(a, w[...], preferred_element_type=jnp.float32)
             + jnp.dot(h, ws[...], preferred_element_type=jnp.float32)
             + bb[...])
        return jnp.maximum(y, 0.0)

    h = layer(agg, x, w0, ws0, bb0)
    h = layer(agg, h, w1, ws1, bb1)
    h = layer(agg, h, w2, ws2, bb2)

    @pl.when(i == 0)
    def _():
        pooled[...] = jnp.full((NG, D), -jnp.inf, dtype=jnp.float32)

    bcol = b3_ref[0]
    for g in range(NG):
        m = (bcol == g)
        val = jnp.max(jnp.where(m, h, -jnp.inf), axis=0, keepdims=True)
        pooled[g:g + 1, :] = jnp.maximum(pooled[g:g + 1, :], val)

    @pl.when(i == NBLK - 1)
    def _():
        out_ref[...] = (jnp.dot(pooled[...], wl[...],
                                preferred_element_type=jnp.float32) + bl[...])


def kernel(x_points, x_clusters, edge_src, edge_dst, batch,
           W_lin, b_lin,
           W_nbr_0, W_self_0, b_0,
           W_nbr_1, W_self_1, b_1,
           W_nbr_2, W_self_2, b_2):
    f32 = jnp.float32
    i32 = jnp.int32

    tile_of = (edge_dst // RANGE).astype(i32)
    order = jnp.argsort(tile_of, stable=True)
    es_s = edge_src[order]
    ed_s = edge_dst[order]
    ts = tile_of[order]
    starts = jnp.searchsorted(ts, jnp.arange(NW, dtype=i32)).astype(i32)
    pos = jnp.arange(E, dtype=i32) - starts[ts] + ts * CAP
    es3 = jnp.full((NW * CAP,), ZROW, i32).at[pos].set(es_s).reshape(NW, NCH, CHUNK)
    slot = jnp.arange(NW * CAP, dtype=i32)
    pad_dst = (slot // CAP) * RANGE + (slot % CAP) % RANGE
    ed3 = pad_dst.at[pos].set(ed_s).reshape(NW, NCH, CHUNK)

    count_t = jnp.diff(jnp.concatenate([starts, jnp.array([E], i32)]))
    cloc = jnp.arange(RANGE, dtype=i32)
    first = count_t[:, None] + (cloc[None, :] - count_t[:, None]) % RANGE
    pad_cnt = jnp.where(first < CAP, (CAP - 1 - first) // RANGE + 1, 0)
    pad_cnt3 = pad_cnt.astype(f32).reshape(NBLK, RB, 1)

    xz = jnp.concatenate([x_points, jnp.zeros((8, D), f32)])
    zsum = jnp.zeros((N_ROWS, D), f32)

    mesh = plsc.VectorSubcoreMesh(core_axis_name="c", subcore_axis_name="s")
    sc_agg = pl.kernel(
        _sc_edge_aggregate_body,
        out_type=(jax.ShapeDtypeStruct((SC_CORES, N_ROWS, D), f32),
                  jax.ShapeDtypeStruct((SC_CORES, N_ROWS, D), f32)),
        mesh=mesh,
        scratch_types=[
            pltpu.VMEM((NCH, CHUNK), i32),
            pltpu.VMEM((NCH, CHUNK), i32),
            pltpu.VMEM((CHUNK, D), f32),
            pltpu.VMEM((CHUNK,), i32),
            pltpu.VMEM((CHUNK, D), f32),
            pltpu.VMEM_SHARED((N_ROWS, D), f32),
            pltpu.SemaphoreType.DMA,
        ],
    )
    sums, cnts = sc_agg(xz, es3, ed3, zsum)

    xc_pad = jnp.concatenate([x_clusters, jnp.zeros((N_ROWS - N_CL, D), f32)])
    batch3 = jnp.concatenate([batch, jnp.full((N_ROWS - N_CL,), NG, i32)]
                             ).reshape(NBLK, RB, 1)
    wl_pad = jnp.concatenate([W_lin, jnp.zeros((D, NG - NC_OUT), f32)], axis=1)
    bl_pad = jnp.concatenate([b_lin, jnp.zeros((NG - NC_OUT,), f32)]).reshape(1, NG)

    full = lambda shape: pl.BlockSpec(shape, lambda i: (0,) * len(shape))
    out16 = pl.pallas_call(
        _tc_body,
        grid=(NBLK,),
        in_specs=[
            pl.BlockSpec((SC_CORES, RB, D), lambda i: (0, i, 0)),
            pl.BlockSpec((SC_CORES, RB, D), lambda i: (0, i, 0)),
            pl.BlockSpec((1, RB, 1), lambda i: (i, 0, 0)),
            pl.BlockSpec((RB, D), lambda i: (i, 0)),
            pl.BlockSpec((1, RB, 1), lambda i: (i, 0, 0)),
            full((D, D)), full((D, D)), full((1, D)),
            full((D, D)), full((D, D)), full((1, D)),
            full((D, D)), full((D, D)), full((1, D)),
            full((D, NG)), full((1, NG)),
        ],
        out_specs=pl.BlockSpec((NG, NG), lambda i: (0, 0)),
        out_shape=jax.ShapeDtypeStruct((NG, NG), f32),
        scratch_shapes=[pltpu.VMEM((NG, D), f32)],
    )(sums, cnts, pad_cnt3, xc_pad, batch3,
      W_nbr_0, W_self_0, b_0.reshape(1, D),
      W_nbr_1, W_self_1, b_1.reshape(1, D),
      W_nbr_2, W_self_2, b_2.reshape(1, D),
      wl_pad, bl_pad)
    return out16[:, :NC_OUT]
